# baseline (speedup 1.0000x reference)
import functools

import jax
import jax.numpy as jnp
from jax import lax
from jax.experimental import pallas as pl
from jax.experimental.pallas import tpu as pltpu
from jax.experimental.pallas import tpu_sc as plsc

N = 10000
NPAD = 10240
E = 320000
H = 128
G = 512
GPAD = 1024

NC, NS, L = 2, 16, 16
NW = NC * NS

EC = 125
ER = E // EC
ERW = ER // NW
ZC = 64
ZR = NPAD // ZC
ZRW = ZR // NW

_MESH = plsc.VectorSubcoreMesh(core_axis_name="c", subcore_axis_name="s")
_F32 = jnp.float32


def _zero_vec(ref, n):
    for i in range(n // L):
        ref[pl.ds(i * L, L)] = jnp.zeros((L,), _F32)


def _prep_body(src2d, dst2d, batch3d, z3d, ztab,
               x0_out, degp_out, cntp_out,
               src_buf, dst_buf, batch_buf, z_buf, ones_v, zeros_v, rows_v,
               deg_sh, cnt_sh, sem):
    cid = lax.axis_index("c")
    sid = lax.axis_index("s")
    wid = cid * NS + sid

    _zero_vec(zeros_v, 640)
    for i in range(128 // L):
        ones_v[pl.ds(i * L, L)] = jnp.ones((L,), _F32)

    pltpu.sync_copy(zeros_v, deg_sh.at[pl.ds(sid * 640, 640)])
    pltpu.sync_copy(zeros_v.at[pl.ds(0, GPAD // NS)],
                    cnt_sh.at[pl.ds(sid * (GPAD // NS), GPAD // NS)])

    pltpu.sync_copy(src2d.at[pl.ds(wid * ERW, ERW)], src_buf)
    pltpu.sync_copy(dst2d.at[pl.ds(wid * ERW, ERW)], dst_buf)
    pltpu.sync_copy(batch3d.at[wid], batch_buf)
    pltpu.sync_copy(z3d.at[wid], z_buf)

    plsc.subcore_barrier()

    def deg_step(j, _):
        pltpu.sync_copy(ones_v.at[pl.ds(0, EC)], deg_sh.at[dst_buf.at[j]],
                        add=True)
        return _
    lax.fori_loop(0, ERW, deg_step, None)

    def cnt_step(j, _):
        pltpu.sync_copy(ones_v.at[pl.ds(0, ZC)], cnt_sh.at[batch_buf.at[j]],
                        add=True)
        return _
    lax.fori_loop(0, ZRW, cnt_step, None)

    for j in range(ZRW):
        pltpu.async_copy(ztab.at[z_buf.at[j]], rows_v, sem).wait()
        pltpu.sync_copy(rows_v, x0_out.at[pl.ds((wid * ZRW + j) * ZC, ZC)])

    plsc.subcore_barrier()

    pltpu.sync_copy(deg_sh.at[pl.ds(sid * 640, 640)], zeros_v)
    pltpu.sync_copy(zeros_v, degp_out.at[cid, pl.ds(sid * 640, 640)])
    cslc = pl.ds(sid * (GPAD // NS), GPAD // NS)
    pltpu.sync_copy(cnt_sh.at[cslc], ones_v.at[pl.ds(0, GPAD // NS)])
    pltpu.sync_copy(ones_v.at[pl.ds(0, GPAD // NS)], cntp_out.at[cid, cslc])


_prep = functools.partial(
    pl.kernel, _prep_body,
    out_type=(jax.ShapeDtypeStruct((NPAD, H), _F32),
              jax.ShapeDtypeStruct((NC, NPAD), _F32),
              jax.ShapeDtypeStruct((NC, GPAD), _F32)),
    mesh=_MESH,
    scratch_types=(pltpu.VMEM((ERW, EC), jnp.int32),
                   pltpu.VMEM((ERW, EC), jnp.int32),
                   pltpu.VMEM((ZRW, ZC), jnp.int32),
                   pltpu.VMEM((ZRW, ZC), jnp.int32),
                   pltpu.VMEM((128,), _F32),
                   pltpu.VMEM((640,), _F32),
                   pltpu.VMEM((ZC, H), _F32),
                   pltpu.VMEM_SHARED((NPAD,), _F32),
                   pltpu.VMEM_SHARED((GPAD,), _F32),
                   pltpu.SemaphoreType.DMA),
)


def _spmm_body(src2d, dst2d, hp,
               sp_out,
               src_buf, dst_buf, zrow_v, rows_a, rows_b, s_sh, sem_a, sem_b):
    cid = lax.axis_index("c")
    sid = lax.axis_index("s")
    wid = cid * NS + sid

    for i in range(L):
        for c in range(H // L):
            zrow_v[i, pl.ds(c * L, L)] = jnp.zeros((L,), _F32)

    for t in range(640 // L):
        pltpu.sync_copy(zrow_v, s_sh.at[pl.ds(sid * 640 + L * t, L)])

    plsc.subcore_barrier()

    EH = ERW // 2
    for half in range(2):
        pltpu.sync_copy(src2d.at[pl.ds(wid * ERW + half * EH, EH)], src_buf)
        pltpu.sync_copy(dst2d.at[pl.ds(wid * ERW + half * EH, EH)], dst_buf)
        pltpu.async_copy(hp.at[src_buf.at[0]], rows_a, sem_a)

        def step2(t, _):
            j0 = 2 * t
            pltpu.make_async_copy(hp.at[src_buf.at[j0]], rows_a, sem_a).wait()
            pltpu.async_copy(hp.at[src_buf.at[j0 + 1]], rows_b, sem_b)
            pltpu.sync_copy(rows_a, s_sh.at[dst_buf.at[j0]], add=True)
            pltpu.make_async_copy(hp.at[src_buf.at[j0 + 1]], rows_b,
                                  sem_b).wait()

            @pl.when(t < EH // 2 - 1)
            def _start_next():
                pltpu.async_copy(hp.at[src_buf.at[j0 + 2]], rows_a, sem_a)

            pltpu.sync_copy(rows_b, s_sh.at[dst_buf.at[j0 + 1]], add=True)
            return _
        lax.fori_loop(0, EH // 2, step2, None)

    plsc.subcore_barrier()

    pltpu.sync_copy(s_sh.at[pl.ds(sid * 640, 640)],
                    sp_out.at[cid, pl.ds(sid * 640, 640)])


_spmm = functools.partial(
    pl.kernel, _spmm_body,
    out_type=jax.ShapeDtypeStruct((NC, NPAD, H), _F32),
    mesh=_MESH,
    scratch_types=(pltpu.VMEM((ERW // 2, EC), jnp.int32),
                   pltpu.VMEM((ERW // 2, EC), jnp.int32),
                   pltpu.VMEM((L, H), _F32),
                   pltpu.VMEM((EC, H), _F32),
                   pltpu.VMEM((EC, H), _F32),
                   pltpu.VMEM_SHARED((NPAD, H), _F32),
                   pltpu.SemaphoreType.DMA,
                   pltpu.SemaphoreType.DMA),
)


GW = G // NW


def _readout_body(sp, hp, dis, sidx, didx, u_out, v_out, dsp_out,
                  si_v, di_v, s0a, s1a, hpa, s0b, s1b, hpb,
                  dsa, dsb, sem):
    cid = lax.axis_index("c")
    sid = lax.axis_index("s")
    wid = cid * NS + sid
    base = wid * GW

    pltpu.sync_copy(sidx.at[pl.ds(base, GW)], si_v)
    pltpu.sync_copy(didx.at[pl.ds(base, GW)], di_v)
    pltpu.async_copy(sp.at[0].at[si_v], s0a, sem).wait()
    pltpu.async_copy(sp.at[1].at[si_v], s1a, sem).wait()
    pltpu.async_copy(hp.at[si_v], hpa, sem).wait()
    pltpu.async_copy(dis.at[si_v], dsa, sem).wait()
    pltpu.async_copy(sp.at[0].at[di_v], s0b, sem).wait()
    pltpu.async_copy(sp.at[1].at[di_v], s1b, sem).wait()
    pltpu.async_copy(hp.at[di_v], hpb, sem).wait()
    pltpu.async_copy(dis.at[di_v], dsb, sem).wait()

    for r in range(GW):
        for c in range(H // L):
            sl = pl.ds(c * L, L)
            s0a[r, sl] = s0a[r, sl] + s1a[r, sl] + hpa[r, sl]
            s0b[r, sl] = s0b[r, sl] + s1b[r, sl] + hpb[r, sl]
    pltpu.sync_copy(s0a, u_out.at[pl.ds(base, GW)])
    pltpu.sync_copy(s0b, v_out.at[pl.ds(base, GW)])
    pltpu.sync_copy(dsa, dsp_out.at[0, pl.ds(base, GW)])
    pltpu.sync_copy(dsb, dsp_out.at[1, pl.ds(base, GW)])


_readout = functools.partial(
    pl.kernel, _readout_body,
    out_type=(jax.ShapeDtypeStruct((G, H), _F32),
              jax.ShapeDtypeStruct((G, H), _F32),
              jax.ShapeDtypeStruct((2, G), _F32)),
    mesh=_MESH,
    scratch_types=(pltpu.VMEM((GW,), jnp.int32),
                   pltpu.VMEM((GW,), jnp.int32),
                   pltpu.VMEM((GW, H), _F32),
                   pltpu.VMEM((GW, H), _F32),
                   pltpu.VMEM((GW, H), _F32),
                   pltpu.VMEM((GW, H), _F32),
                   pltpu.VMEM((GW, H), _F32),
                   pltpu.VMEM((GW, H), _F32),
                   pltpu.VMEM((GW,), _F32),
                   pltpu.VMEM((GW,), _F32),
                   pltpu.SemaphoreType.DMA),
)


_PREC = lax.Precision.DEFAULT


def _tc1_body(x0_ref, w_ref, degp_ref, cntp_ref, hp_ref, dis_ref, cidx_ref):
    degp = degp_ref[...]
    dis = 1.0 / jnp.sqrt(degp[0] + degp[1] + 1.0)
    dis_ref[...] = dis
    h = jnp.dot(x0_ref[...], w_ref[...], precision=_PREC,
                preferred_element_type=_F32)
    hp_ref[...] = h * dis
    cnt = cntp_ref[...]
    c = cnt[0:1] + cnt[1:2]
    inc = c
    k = 1
    while k < GPAD:
        inc = inc + jnp.concatenate(
            [jnp.zeros((1, k), _F32), inc[:, :GPAD - k]], axis=1)
        k *= 2
    ex = inc - c
    center = jnp.where(c > 0, ex, float(N)).astype(jnp.int32)
    s = jnp.minimum(center, N - 1)
    d = jnp.minimum(center + 1, N - 1)
    cidx_ref[...] = jnp.concatenate([s, d], axis=0)


def _tc1(x0, w0, degp, cntp):
    return pl.pallas_call(
        _tc1_body,
        out_shape=(jax.ShapeDtypeStruct((NPAD, H), _F32),
                   jax.ShapeDtypeStruct((NPAD, 1), _F32),
                   jax.ShapeDtypeStruct((2, GPAD), jnp.int32)),
    )(x0, w0, degp, cntp)


def _tc2_body(sp_ref, hp_ref, dis_ref, b_ref, w_ref, out_ref):
    dis = dis_ref[...]
    s = sp_ref[0] + sp_ref[1] + hp_ref[...]
    x = jnp.maximum(dis * s + b_ref[...], 0.0)
    out_ref[...] = jnp.dot(x, w_ref[...], precision=_PREC,
                           preferred_element_type=_F32) * dis


def _tc2(sp, hp, dis, b, w):
    return pl.pallas_call(
        _tc2_body,
        out_shape=jax.ShapeDtypeStruct((NPAD, H), _F32),
    )(sp, hp, dis, b, w)


def _tc4_body(u_ref, v_ref, dsp_ref, bl_ref, w1_ref, b1_ref, w2_ref,
              b2_ref, out_ref):
    dsp = dsp_ref[...]
    bl = bl_ref[...]
    h = (dsp[0] * u_ref[...] + bl) * (dsp[1] * v_ref[...] + bl)
    t = jnp.maximum(jnp.dot(h, w1_ref[...], precision=_PREC,
                            preferred_element_type=_F32) + b1_ref[...], 0.0)
    out_ref[...] = jnp.dot(t, w2_ref[...], precision=_PREC,
                           preferred_element_type=_F32) + b2_ref[...]


def _tc4(u, v, dsp, bl, w1, b1, w2, b2):
    return pl.pallas_call(
        _tc4_body,
        out_shape=jax.ShapeDtypeStruct((G, 1), _F32),
    )(u, v, dsp, bl, w1, b1, w2, b2)


def kernel(num_nodes, z, edge_index, batch, z_table,
           W0, b0, W1, b1, W2, b2, mw1, mb1, mw2, mb2):
    src2d = edge_index[0].astype(jnp.int32).reshape(ER, EC)
    dst2d = edge_index[1].astype(jnp.int32).reshape(ER, EC)
    batch3d = jnp.pad(batch.astype(jnp.int32), (0, NPAD - N),
                      constant_values=G).reshape(NW, ZRW, ZC)
    z3d = jnp.pad(z.astype(jnp.int32), (0, NPAD - N)).reshape(NW, ZRW, ZC)

    x0, degp, cntp = _prep()(src2d, dst2d, batch3d, z3d, z_table)

    hp0, dis, cidx = _tc1(x0, W0, degp.reshape(NC, NPAD, 1), cntp)
    sp0 = _spmm()(src2d, dst2d, hp0)
    hp1 = _tc2(sp0, hp0, dis, b0.reshape(1, H), W1)
    sp1 = _spmm()(src2d, dst2d, hp1)
    hp2 = _tc2(sp1, hp1, dis, b1.reshape(1, H), W2)
    sp2 = _spmm()(src2d, dst2d, hp2)

    u, v, dsp = _readout()(sp2, hp2, dis.reshape(NPAD),
                           cidx[0, :G], cidx[1, :G])
    return _tc4(u, v, dsp.reshape(2, G, 1), b2.reshape(1, H),
                mw1, mb1.reshape(1, H), mw2, mb2.reshape(1, 1))


# speedup vs baseline: 1.0265x; 1.0265x over previous
import functools

import jax
import jax.numpy as jnp
from jax import lax
from jax.experimental import pallas as pl
from jax.experimental.pallas import tpu as pltpu
from jax.experimental.pallas import tpu_sc as plsc

N = 10000
NPAD = 10240
E = 320000
H = 128
G = 512
GPAD = 1024

NC, NS, L = 2, 16, 16
NW = NC * NS

EC = 125
ER = E // EC
ERW = ER // NW
ZC = 64
ZR = NPAD // ZC
ZRW = ZR // NW

_MESH = plsc.VectorSubcoreMesh(core_axis_name="c", subcore_axis_name="s")
_F32 = jnp.float32


def _zero_vec(ref, n):
    for i in range(n // L):
        ref[pl.ds(i * L, L)] = jnp.zeros((L,), _F32)


def _prep_body(src2d, dst2d, batch3d, z3d, ztab,
               x0_out, degp_out, cntp_out,
               src_buf, dst_buf, batch_buf, z_buf, ones_v, zeros_v, rows_v,
               deg_sh, cnt_sh, sem):
    cid = lax.axis_index("c")
    sid = lax.axis_index("s")
    wid = cid * NS + sid

    _zero_vec(zeros_v, 640)
    for i in range(128 // L):
        ones_v[pl.ds(i * L, L)] = jnp.ones((L,), _F32)

    pltpu.sync_copy(zeros_v, deg_sh.at[pl.ds(sid * 640, 640)])
    pltpu.sync_copy(zeros_v.at[pl.ds(0, GPAD // NS)],
                    cnt_sh.at[pl.ds(sid * (GPAD // NS), GPAD // NS)])

    pltpu.sync_copy(src2d.at[pl.ds(wid * ERW, ERW)], src_buf)
    pltpu.sync_copy(dst2d.at[pl.ds(wid * ERW, ERW)], dst_buf)
    pltpu.sync_copy(batch3d.at[wid], batch_buf)
    pltpu.sync_copy(z3d.at[wid], z_buf)

    plsc.subcore_barrier()

    def deg_step(j, _):
        pltpu.sync_copy(ones_v.at[pl.ds(0, EC)], deg_sh.at[dst_buf.at[j]],
                        add=True)
        return _
    lax.fori_loop(0, ERW, deg_step, None)

    def cnt_step(j, _):
        pltpu.sync_copy(ones_v.at[pl.ds(0, ZC)], cnt_sh.at[batch_buf.at[j]],
                        add=True)
        return _
    lax.fori_loop(0, ZRW, cnt_step, None)

    for j in range(ZRW):
        pltpu.async_copy(ztab.at[z_buf.at[j]], rows_v, sem).wait()
        pltpu.sync_copy(rows_v, x0_out.at[pl.ds((wid * ZRW + j) * ZC, ZC)])

    plsc.subcore_barrier()

    pltpu.sync_copy(deg_sh.at[pl.ds(sid * 640, 640)], zeros_v)
    pltpu.sync_copy(zeros_v, degp_out.at[cid, pl.ds(sid * 640, 640)])
    cslc = pl.ds(sid * (GPAD // NS), GPAD // NS)
    pltpu.sync_copy(cnt_sh.at[cslc], ones_v.at[pl.ds(0, GPAD // NS)])
    pltpu.sync_copy(ones_v.at[pl.ds(0, GPAD // NS)], cntp_out.at[cid, cslc])


_prep = functools.partial(
    pl.kernel, _prep_body,
    out_type=(jax.ShapeDtypeStruct((NPAD, H), _F32),
              jax.ShapeDtypeStruct((NC, NPAD), _F32),
              jax.ShapeDtypeStruct((NC, GPAD), _F32)),
    mesh=_MESH,
    scratch_types=(pltpu.VMEM((ERW, EC), jnp.int32),
                   pltpu.VMEM((ERW, EC), jnp.int32),
                   pltpu.VMEM((ZRW, ZC), jnp.int32),
                   pltpu.VMEM((ZRW, ZC), jnp.int32),
                   pltpu.VMEM((128,), _F32),
                   pltpu.VMEM((640,), _F32),
                   pltpu.VMEM((ZC, H), _F32),
                   pltpu.VMEM_SHARED((NPAD,), _F32),
                   pltpu.VMEM_SHARED((GPAD,), _F32),
                   pltpu.SemaphoreType.DMA),
)


def _spmm_body(src2d, dst2d, hp,
               sp_out,
               src_buf, dst_buf, zrow_v, rows_a, rows_b, s_sh,
               sem_a, sem_a2, sem_b, sem_b2):
    cid = lax.axis_index("c")
    sid = lax.axis_index("s")
    wid = cid * NS + sid

    for i in range(L):
        for c in range(H // L):
            zrow_v[i, pl.ds(c * L, L)] = jnp.zeros((L,), _F32)

    for t in range(640 // L):
        pltpu.sync_copy(zrow_v, s_sh.at[pl.ds(sid * 640 + L * t, L)])

    plsc.subcore_barrier()

    EH = ERW // 2
    for half in range(2):
        pltpu.sync_copy(src2d.at[pl.ds(wid * ERW + half * EH, EH)], src_buf)
        pltpu.sync_copy(dst2d.at[pl.ds(wid * ERW + half * EH, EH)], dst_buf)
        EC0 = 64

        def start_gather(j, rows, s1, s2):
            pltpu.async_copy(hp.at[src_buf.at[j, pl.ds(0, EC0)]],
                             rows.at[pl.ds(0, EC0)], s1)
            pltpu.async_copy(hp.at[src_buf.at[j, pl.ds(EC0, EC - EC0)]],
                             rows.at[pl.ds(EC0, EC - EC0)], s2)

        def wait_gather(j, rows, s1, s2):
            pltpu.make_async_copy(hp.at[src_buf.at[j, pl.ds(0, EC0)]],
                                  rows.at[pl.ds(0, EC0)], s1).wait()
            pltpu.make_async_copy(hp.at[src_buf.at[j, pl.ds(EC0, EC - EC0)]],
                                  rows.at[pl.ds(EC0, EC - EC0)], s2).wait()

        start_gather(0, rows_a, sem_a, sem_a2)

        def step2(t, _):
            j0 = 2 * t
            wait_gather(j0, rows_a, sem_a, sem_a2)
            start_gather(j0 + 1, rows_b, sem_b, sem_b2)
            pltpu.sync_copy(rows_a, s_sh.at[dst_buf.at[j0]], add=True)
            wait_gather(j0 + 1, rows_b, sem_b, sem_b2)

            @pl.when(t < EH // 2 - 1)
            def _start_next():
                start_gather(j0 + 2, rows_a, sem_a, sem_a2)

            pltpu.sync_copy(rows_b, s_sh.at[dst_buf.at[j0 + 1]], add=True)
            return _
        lax.fori_loop(0, EH // 2, step2, None)

    plsc.subcore_barrier()

    pltpu.sync_copy(s_sh.at[pl.ds(sid * 640, 640)],
                    sp_out.at[cid, pl.ds(sid * 640, 640)])


_spmm = functools.partial(
    pl.kernel, _spmm_body,
    out_type=jax.ShapeDtypeStruct((NC, NPAD, H), _F32),
    mesh=_MESH,
    scratch_types=(pltpu.VMEM((ERW // 2, EC), jnp.int32),
                   pltpu.VMEM((ERW // 2, EC), jnp.int32),
                   pltpu.VMEM((L, H), _F32),
                   pltpu.VMEM((EC, H), _F32),
                   pltpu.VMEM((EC, H), _F32),
                   pltpu.VMEM_SHARED((NPAD, H), _F32),
                   pltpu.SemaphoreType.DMA,
                   pltpu.SemaphoreType.DMA,
                   pltpu.SemaphoreType.DMA,
                   pltpu.SemaphoreType.DMA),
)


GW = G // NW


def _readout_body(sp, hp, dis, sidx, didx, u_out, v_out, dsp_out,
                  si_v, di_v, s0a, s1a, hpa, s0b, s1b, hpb,
                  dsa, dsb, sem):
    cid = lax.axis_index("c")
    sid = lax.axis_index("s")
    wid = cid * NS + sid
    base = wid * GW

    pltpu.sync_copy(sidx.at[pl.ds(base, GW)], si_v)
    pltpu.sync_copy(didx.at[pl.ds(base, GW)], di_v)
    pltpu.async_copy(sp.at[0].at[si_v], s0a, sem).wait()
    pltpu.async_copy(sp.at[1].at[si_v], s1a, sem).wait()
    pltpu.async_copy(hp.at[si_v], hpa, sem).wait()
    pltpu.async_copy(dis.at[si_v], dsa, sem).wait()
    pltpu.async_copy(sp.at[0].at[di_v], s0b, sem).wait()
    pltpu.async_copy(sp.at[1].at[di_v], s1b, sem).wait()
    pltpu.async_copy(hp.at[di_v], hpb, sem).wait()
    pltpu.async_copy(dis.at[di_v], dsb, sem).wait()

    for r in range(GW):
        for c in range(H // L):
            sl = pl.ds(c * L, L)
            s0a[r, sl] = s0a[r, sl] + s1a[r, sl] + hpa[r, sl]
            s0b[r, sl] = s0b[r, sl] + s1b[r, sl] + hpb[r, sl]
    pltpu.sync_copy(s0a, u_out.at[pl.ds(base, GW)])
    pltpu.sync_copy(s0b, v_out.at[pl.ds(base, GW)])
    pltpu.sync_copy(dsa, dsp_out.at[0, pl.ds(base, GW)])
    pltpu.sync_copy(dsb, dsp_out.at[1, pl.ds(base, GW)])


_readout = functools.partial(
    pl.kernel, _readout_body,
    out_type=(jax.ShapeDtypeStruct((G, H), _F32),
              jax.ShapeDtypeStruct((G, H), _F32),
              jax.ShapeDtypeStruct((2, G), _F32)),
    mesh=_MESH,
    scratch_types=(pltpu.VMEM((GW,), jnp.int32),
                   pltpu.VMEM((GW,), jnp.int32),
                   pltpu.VMEM((GW, H), _F32),
                   pltpu.VMEM((GW, H), _F32),
                   pltpu.VMEM((GW, H), _F32),
                   pltpu.VMEM((GW, H), _F32),
                   pltpu.VMEM((GW, H), _F32),
                   pltpu.VMEM((GW, H), _F32),
                   pltpu.VMEM((GW,), _F32),
                   pltpu.VMEM((GW,), _F32),
                   pltpu.SemaphoreType.DMA),
)


_PREC = lax.Precision.DEFAULT


def _tc1_body(x0_ref, w_ref, degp_ref, cntp_ref, hp_ref, dis_ref, cidx_ref):
    degp = degp_ref[...]
    dis = 1.0 / jnp.sqrt(degp[0] + degp[1] + 1.0)
    dis_ref[...] = dis
    h = jnp.dot(x0_ref[...], w_ref[...], precision=_PREC,
                preferred_element_type=_F32)
    hp_ref[...] = h * dis
    cnt = cntp_ref[...]
    c = cnt[0:1] + cnt[1:2]
    inc = c
    k = 1
    while k < GPAD:
        inc = inc + jnp.concatenate(
            [jnp.zeros((1, k), _F32), inc[:, :GPAD - k]], axis=1)
        k *= 2
    ex = inc - c
    center = jnp.where(c > 0, ex, float(N)).astype(jnp.int32)
    s = jnp.minimum(center, N - 1)
    d = jnp.minimum(center + 1, N - 1)
    cidx_ref[...] = jnp.concatenate([s, d], axis=0)


def _tc1(x0, w0, degp, cntp):
    return pl.pallas_call(
        _tc1_body,
        out_shape=(jax.ShapeDtypeStruct((NPAD, H), _F32),
                   jax.ShapeDtypeStruct((NPAD, 1), _F32),
                   jax.ShapeDtypeStruct((2, GPAD), jnp.int32)),
    )(x0, w0, degp, cntp)


def _tc2_body(sp_ref, hp_ref, dis_ref, b_ref, w_ref, out_ref):
    dis = dis_ref[...]
    s = sp_ref[0] + sp_ref[1] + hp_ref[...]
    x = jnp.maximum(dis * s + b_ref[...], 0.0)
    out_ref[...] = jnp.dot(x, w_ref[...], precision=_PREC,
                           preferred_element_type=_F32) * dis


def _tc2(sp, hp, dis, b, w):
    return pl.pallas_call(
        _tc2_body,
        out_shape=jax.ShapeDtypeStruct((NPAD, H), _F32),
    )(sp, hp, dis, b, w)


def _tc4_body(u_ref, v_ref, dsp_ref, bl_ref, w1_ref, b1_ref, w2_ref,
              b2_ref, out_ref):
    dsp = dsp_ref[...]
    bl = bl_ref[...]
    h = (dsp[0] * u_ref[...] + bl) * (dsp[1] * v_ref[...] + bl)
    t = jnp.maximum(jnp.dot(h, w1_ref[...], precision=_PREC,
                            preferred_element_type=_F32) + b1_ref[...], 0.0)
    out_ref[...] = jnp.dot(t, w2_ref[...], precision=_PREC,
                           preferred_element_type=_F32) + b2_ref[...]


def _tc4(u, v, dsp, bl, w1, b1, w2, b2):
    return pl.pallas_call(
        _tc4_body,
        out_shape=jax.ShapeDtypeStruct((G, 1), _F32),
    )(u, v, dsp, bl, w1, b1, w2, b2)


def kernel(num_nodes, z, edge_index, batch, z_table,
           W0, b0, W1, b1, W2, b2, mw1, mb1, mw2, mb2):
    src2d = edge_index[0].astype(jnp.int32).reshape(ER, EC)
    dst2d = edge_index[1].astype(jnp.int32).reshape(ER, EC)
    batch3d = jnp.pad(batch.astype(jnp.int32), (0, NPAD - N),
                      constant_values=G).reshape(NW, ZRW, ZC)
    z3d = jnp.pad(z.astype(jnp.int32), (0, NPAD - N)).reshape(NW, ZRW, ZC)

    x0, degp, cntp = _prep()(src2d, dst2d, batch3d, z3d, z_table)

    hp0, dis, cidx = _tc1(x0, W0, degp.reshape(NC, NPAD, 1), cntp)
    sp0 = _spmm()(src2d, dst2d, hp0)
    hp1 = _tc2(sp0, hp0, dis, b0.reshape(1, H), W1)
    sp1 = _spmm()(src2d, dst2d, hp1)
    hp2 = _tc2(sp1, hp1, dis, b1.reshape(1, H), W2)
    sp2 = _spmm()(src2d, dst2d, hp2)

    u, v, dsp = _readout()(sp2, hp2, dis.reshape(NPAD),
                           cidx[0, :G], cidx[1, :G])
    return _tc4(u, v, dsp.reshape(2, G, 1), b2.reshape(1, H),
                mw1, mb1.reshape(1, H), mw2, mb2.reshape(1, 1))


# speedup vs baseline: 1.1282x; 1.0990x over previous
import functools

import jax
import jax.numpy as jnp
from jax import lax
from jax.experimental import pallas as pl
from jax.experimental.pallas import tpu as pltpu
from jax.experimental.pallas import tpu_sc as plsc

N = 10000
NPAD = 10240
E = 320000
H = 128
G = 512
GPAD = 1024

NC, NS, L = 2, 16, 16
NW = NC * NS

EC = 125
ER = E // EC
ERW = ER // NW
ZC = 64
ZR = NPAD // ZC
ZRW = ZR // NW

_MESH = plsc.VectorSubcoreMesh(core_axis_name="c", subcore_axis_name="s")
_F32 = jnp.float32


def _zero_vec(ref, n):
    for i in range(n // L):
        ref[pl.ds(i * L, L)] = jnp.zeros((L,), _F32)


def _prep_body(src2d, dst2d, batch3d, z3d, ztab,
               x0_out, degp_out, cntp_out,
               src_buf, dst_buf, batch_buf, z_buf, ones_v, zeros_v, rows_v,
               deg_sh, cnt_sh, sem):
    cid = lax.axis_index("c")
    sid = lax.axis_index("s")
    wid = cid * NS + sid

    _zero_vec(zeros_v, 640)
    for i in range(128 // L):
        ones_v[pl.ds(i * L, L)] = jnp.ones((L,), _F32)

    pltpu.sync_copy(zeros_v, deg_sh.at[pl.ds(sid * 640, 640)])
    pltpu.sync_copy(zeros_v.at[pl.ds(0, GPAD // NS)],
                    cnt_sh.at[pl.ds(sid * (GPAD // NS), GPAD // NS)])

    pltpu.sync_copy(src2d.at[pl.ds(wid * ERW, ERW)], src_buf)
    pltpu.sync_copy(dst2d.at[pl.ds(wid * ERW, ERW)], dst_buf)
    pltpu.sync_copy(batch3d.at[wid], batch_buf)
    pltpu.sync_copy(z3d.at[wid], z_buf)

    plsc.subcore_barrier()

    def deg_step(j, _):
        pltpu.sync_copy(ones_v.at[pl.ds(0, EC)], deg_sh.at[dst_buf.at[j]],
                        add=True)
        return _
    lax.fori_loop(0, ERW, deg_step, None)

    def cnt_step(j, _):
        pltpu.sync_copy(ones_v.at[pl.ds(0, ZC)], cnt_sh.at[batch_buf.at[j]],
                        add=True)
        return _
    lax.fori_loop(0, ZRW, cnt_step, None)

    for j in range(ZRW):
        pltpu.async_copy(ztab.at[z_buf.at[j]], rows_v, sem).wait()
        pltpu.sync_copy(rows_v, x0_out.at[pl.ds((wid * ZRW + j) * ZC, ZC)])

    plsc.subcore_barrier()

    pltpu.sync_copy(deg_sh.at[pl.ds(sid * 640, 640)], zeros_v)
    pltpu.sync_copy(zeros_v, degp_out.at[cid, pl.ds(sid * 640, 640)])
    cslc = pl.ds(sid * (GPAD // NS), GPAD // NS)
    pltpu.sync_copy(cnt_sh.at[cslc], ones_v.at[pl.ds(0, GPAD // NS)])
    pltpu.sync_copy(ones_v.at[pl.ds(0, GPAD // NS)], cntp_out.at[cid, cslc])


_prep = functools.partial(
    pl.kernel, _prep_body,
    out_type=(jax.ShapeDtypeStruct((NPAD, H), _F32),
              jax.ShapeDtypeStruct((NC, NPAD), _F32),
              jax.ShapeDtypeStruct((NC, GPAD), _F32)),
    mesh=_MESH,
    scratch_types=(pltpu.VMEM((ERW, EC), jnp.int32),
                   pltpu.VMEM((ERW, EC), jnp.int32),
                   pltpu.VMEM((ZRW, ZC), jnp.int32),
                   pltpu.VMEM((ZRW, ZC), jnp.int32),
                   pltpu.VMEM((128,), _F32),
                   pltpu.VMEM((640,), _F32),
                   pltpu.VMEM((ZC, H), _F32),
                   pltpu.VMEM_SHARED((NPAD,), _F32),
                   pltpu.VMEM_SHARED((GPAD,), _F32),
                   pltpu.SemaphoreType.DMA),
)


def _spmm_body(src2d, dst2d, hp,
               sp_out,
               src_buf, dst_buf, zrow_v, rows_a, rows_b, s_sh,
               sem_a, sem_a2, sem_b, sem_b2):
    cid = lax.axis_index("c")
    sid = lax.axis_index("s")
    wid = cid * NS + sid

    for i in range(L):
        for c in range(H // L):
            zrow_v[i, pl.ds(c * L, L)] = jnp.zeros((L,), _F32)

    for t in range(640 // L):
        pltpu.sync_copy(zrow_v, s_sh.at[pl.ds(sid * 640 + L * t, L)])

    plsc.subcore_barrier()

    EH = ERW // 2
    for half in range(2):
        pltpu.sync_copy(src2d.at[pl.ds(wid * ERW + half * EH, EH)], src_buf)
        pltpu.sync_copy(dst2d.at[pl.ds(wid * ERW + half * EH, EH)], dst_buf)
        EC0 = 64

        def start_gather(j, rows, s1, s2):
            pltpu.async_copy(hp.at[src_buf.at[j, pl.ds(0, EC0)]],
                             rows.at[pl.ds(0, EC0)], s1)
            pltpu.async_copy(hp.at[src_buf.at[j, pl.ds(EC0, EC - EC0)]],
                             rows.at[pl.ds(EC0, EC - EC0)], s2)

        def wait_gather(j, rows, s1, s2):
            pltpu.make_async_copy(hp.at[src_buf.at[j, pl.ds(0, EC0)]],
                                  rows.at[pl.ds(0, EC0)], s1).wait()
            pltpu.make_async_copy(hp.at[src_buf.at[j, pl.ds(EC0, EC - EC0)]],
                                  rows.at[pl.ds(EC0, EC - EC0)], s2).wait()

        start_gather(0, rows_a, sem_a, sem_a2)
        start_gather(1, rows_b, sem_b, sem_b2)

        def step2(t, _):
            j0 = 2 * t
            wait_gather(j0, rows_a, sem_a, sem_a2)
            pltpu.sync_copy(rows_a, s_sh.at[dst_buf.at[j0]], add=True)

            @pl.when(t < EH // 2 - 1)
            def _next_a():
                start_gather(j0 + 2, rows_a, sem_a, sem_a2)

            wait_gather(j0 + 1, rows_b, sem_b, sem_b2)
            pltpu.sync_copy(rows_b, s_sh.at[dst_buf.at[j0 + 1]], add=True)

            @pl.when(t < EH // 2 - 1)
            def _next_b():
                start_gather(j0 + 3, rows_b, sem_b, sem_b2)

            return _
        lax.fori_loop(0, EH // 2, step2, None)

    plsc.subcore_barrier()

    pltpu.sync_copy(s_sh.at[pl.ds(sid * 640, 640)],
                    sp_out.at[cid, pl.ds(sid * 640, 640)])


_spmm = functools.partial(
    pl.kernel, _spmm_body,
    out_type=jax.ShapeDtypeStruct((NC, NPAD, H), _F32),
    mesh=_MESH,
    scratch_types=(pltpu.VMEM((ERW // 2, EC), jnp.int32),
                   pltpu.VMEM((ERW // 2, EC), jnp.int32),
                   pltpu.VMEM((L, H), _F32),
                   pltpu.VMEM((EC, H), _F32),
                   pltpu.VMEM((EC, H), _F32),
                   pltpu.VMEM_SHARED((NPAD, H), _F32),
                   pltpu.SemaphoreType.DMA,
                   pltpu.SemaphoreType.DMA,
                   pltpu.SemaphoreType.DMA,
                   pltpu.SemaphoreType.DMA),
)


GW = G // NW


def _readout_body(sp, hp, dis, sidx, didx, u_out, v_out, dsp_out,
                  si_v, di_v, s0a, s1a, hpa, s0b, s1b, hpb,
                  dsa, dsb, sem):
    cid = lax.axis_index("c")
    sid = lax.axis_index("s")
    wid = cid * NS + sid
    base = wid * GW

    pltpu.sync_copy(sidx.at[pl.ds(base, GW)], si_v)
    pltpu.sync_copy(didx.at[pl.ds(base, GW)], di_v)
    pltpu.async_copy(sp.at[0].at[si_v], s0a, sem).wait()
    pltpu.async_copy(sp.at[1].at[si_v], s1a, sem).wait()
    pltpu.async_copy(hp.at[si_v], hpa, sem).wait()
    pltpu.async_copy(dis.at[si_v], dsa, sem).wait()
    pltpu.async_copy(sp.at[0].at[di_v], s0b, sem).wait()
    pltpu.async_copy(sp.at[1].at[di_v], s1b, sem).wait()
    pltpu.async_copy(hp.at[di_v], hpb, sem).wait()
    pltpu.async_copy(dis.at[di_v], dsb, sem).wait()

    for r in range(GW):
        for c in range(H // L):
            sl = pl.ds(c * L, L)
            s0a[r, sl] = s0a[r, sl] + s1a[r, sl] + hpa[r, sl]
            s0b[r, sl] = s0b[r, sl] + s1b[r, sl] + hpb[r, sl]
    pltpu.sync_copy(s0a, u_out.at[pl.ds(base, GW)])
    pltpu.sync_copy(s0b, v_out.at[pl.ds(base, GW)])
    pltpu.sync_copy(dsa, dsp_out.at[0, pl.ds(base, GW)])
    pltpu.sync_copy(dsb, dsp_out.at[1, pl.ds(base, GW)])


_readout = functools.partial(
    pl.kernel, _readout_body,
    out_type=(jax.ShapeDtypeStruct((G, H), _F32),
              jax.ShapeDtypeStruct((G, H), _F32),
              jax.ShapeDtypeStruct((2, G), _F32)),
    mesh=_MESH,
    scratch_types=(pltpu.VMEM((GW,), jnp.int32),
                   pltpu.VMEM((GW,), jnp.int32),
                   pltpu.VMEM((GW, H), _F32),
                   pltpu.VMEM((GW, H), _F32),
                   pltpu.VMEM((GW, H), _F32),
                   pltpu.VMEM((GW, H), _F32),
                   pltpu.VMEM((GW, H), _F32),
                   pltpu.VMEM((GW, H), _F32),
                   pltpu.VMEM((GW,), _F32),
                   pltpu.VMEM((GW,), _F32),
                   pltpu.SemaphoreType.DMA),
)


_PREC = lax.Precision.DEFAULT


def _tc1_body(x0_ref, w_ref, degp_ref, cntp_ref, hp_ref, dis_ref, cidx_ref):
    degp = degp_ref[...]
    dis = 1.0 / jnp.sqrt(degp[0] + degp[1] + 1.0)
    dis_ref[...] = dis
    h = jnp.dot(x0_ref[...], w_ref[...], precision=_PREC,
                preferred_element_type=_F32)
    hp_ref[...] = h * dis
    cnt = cntp_ref[...]
    c = cnt[0:1] + cnt[1:2]
    inc = c
    k = 1
    while k < GPAD:
        inc = inc + jnp.concatenate(
            [jnp.zeros((1, k), _F32), inc[:, :GPAD - k]], axis=1)
        k *= 2
    ex = inc - c
    center = jnp.where(c > 0, ex, float(N)).astype(jnp.int32)
    s = jnp.minimum(center, N - 1)
    d = jnp.minimum(center + 1, N - 1)
    cidx_ref[...] = jnp.concatenate([s, d], axis=0)


def _tc1(x0, w0, degp, cntp):
    return pl.pallas_call(
        _tc1_body,
        out_shape=(jax.ShapeDtypeStruct((NPAD, H), _F32),
                   jax.ShapeDtypeStruct((NPAD, 1), _F32),
                   jax.ShapeDtypeStruct((2, GPAD), jnp.int32)),
    )(x0, w0, degp, cntp)


def _tc2_body(sp_ref, hp_ref, dis_ref, b_ref, w_ref, out_ref):
    dis = dis_ref[...]
    s = sp_ref[0] + sp_ref[1] + hp_ref[...]
    x = jnp.maximum(dis * s + b_ref[...], 0.0)
    out_ref[...] = jnp.dot(x, w_ref[...], precision=_PREC,
                           preferred_element_type=_F32) * dis


def _tc2(sp, hp, dis, b, w):
    return pl.pallas_call(
        _tc2_body,
        out_shape=jax.ShapeDtypeStruct((NPAD, H), _F32),
    )(sp, hp, dis, b, w)


def _tc4_body(u_ref, v_ref, dsp_ref, bl_ref, w1_ref, b1_ref, w2_ref,
              b2_ref, out_ref):
    dsp = dsp_ref[...]
    bl = bl_ref[...]
    h = (dsp[0] * u_ref[...] + bl) * (dsp[1] * v_ref[...] + bl)
    t = jnp.maximum(jnp.dot(h, w1_ref[...], precision=_PREC,
                            preferred_element_type=_F32) + b1_ref[...], 0.0)
    out_ref[...] = jnp.dot(t, w2_ref[...], precision=_PREC,
                           preferred_element_type=_F32) + b2_ref[...]


def _tc4(u, v, dsp, bl, w1, b1, w2, b2):
    return pl.pallas_call(
        _tc4_body,
        out_shape=jax.ShapeDtypeStruct((G, 1), _F32),
    )(u, v, dsp, bl, w1, b1, w2, b2)


def kernel(num_nodes, z, edge_index, batch, z_table,
           W0, b0, W1, b1, W2, b2, mw1, mb1, mw2, mb2):
    src2d = edge_index[0].astype(jnp.int32).reshape(ER, EC)
    dst2d = edge_index[1].astype(jnp.int32).reshape(ER, EC)
    batch3d = jnp.pad(batch.astype(jnp.int32), (0, NPAD - N),
                      constant_values=G).reshape(NW, ZRW, ZC)
    z3d = jnp.pad(z.astype(jnp.int32), (0, NPAD - N)).reshape(NW, ZRW, ZC)

    x0, degp, cntp = _prep()(src2d, dst2d, batch3d, z3d, z_table)

    hp0, dis, cidx = _tc1(x0, W0, degp.reshape(NC, NPAD, 1), cntp)
    sp0 = _spmm()(src2d, dst2d, hp0)
    hp1 = _tc2(sp0, hp0, dis, b0.reshape(1, H), W1)
    sp1 = _spmm()(src2d, dst2d, hp1)
    hp2 = _tc2(sp1, hp1, dis, b1.reshape(1, H), W2)
    sp2 = _spmm()(src2d, dst2d, hp2)

    u, v, dsp = _readout()(sp2, hp2, dis.reshape(NPAD),
                           cidx[0, :G], cidx[1, :G])
    return _tc4(u, v, dsp.reshape(2, G, 1), b2.reshape(1, H),
                mw1, mb1.reshape(1, H), mw2, mb2.reshape(1, 1))
